# baseline (speedup 1.0000x reference)
import jax
import jax.numpy as jnp
from jax.experimental import pallas as pl
from jax.experimental.pallas import tpu as pltpu

_H = 32


def _dg(a, b, ca, cb):
    return jax.lax.dot_general(
        a.astype(jnp.bfloat16), b.astype(jnp.bfloat16),
        (((ca,), (cb,)), ((), ())), preferred_element_type=jnp.float32)


def _smg_body(x_ref, a_ref, d_ref, s_ref, o_ref):
    G = x_ref.shape[0]
    n = x_ref.shape[2]

    sT = s_ref
    bcol = sT[:, 288:368]
    l12b0 = bcol[0:64, 0:1]
    m1b0 = bcol[0:32, 8:9]
    m2w0 = bcol[0:32, 16:17]
    m2b0 = bcol[0:1, 24:25]
    l12b1 = bcol[0:64, 32:33]
    m1b1 = bcol[0:32, 40:41]
    m2w1 = bcol[0:32, 48:49]
    m2b1 = bcol[0:1, 56:57]
    p1b = bcol[0:32, 64:65]
    p2b = bcol[0:32, 72:73]

    AT16 = [jnp.swapaxes(a_ref[g][...], 0, 1).astype(jnp.bfloat16)
            for g in range(G)]

    def adots(lhs):
        return jnp.concatenate(
            [_dg(lhs[:, g * n:(g + 1) * n], AT16[g], 1, 0) for g in range(G)],
            axis=1)

    Wp0 = jnp.concatenate([sT[:, 0:16], sT[:, 80:96]], axis=0)
    full0 = jnp.concatenate(
        [_dg(Wp0, x_ref[g], 1, 0) for g in range(G)], axis=1)
    hl0 = full0[0:64] + l12b0
    agg0 = adots(hl0[0:32])
    cat0 = jnp.maximum(jnp.concatenate([agg0, hl0[32:64]], axis=0), 0.0)
    w0 = jnp.maximum(_dg(sT[0:_H, 16:80], cat0, 1, 0) + m1b0, 0.0)
    s0 = jnp.sum(w0 * m2w0, axis=0, keepdims=True) + m2b0
    mask0 = jax.nn.sigmoid(s0)
    h0 = mask0 * full0[64:96]
    x1 = jnp.maximum((adots(h0) + full0[96:128]) * mask0, 0.0)

    Wp1 = jnp.concatenate([sT[:, 96:128], sT[:, 192:224]], axis=0)
    full1 = _dg(Wp1, x1, 1, 0)
    hl1 = full1[0:64] * mask0 + l12b1
    agg1 = adots(hl1[0:32])
    cat1 = jnp.maximum(jnp.concatenate([agg1, hl1[32:64]], axis=0), 0.0)
    w1 = jnp.maximum(_dg(sT[0:_H, 128:192], cat1, 1, 0) + m1b1, 0.0)
    s1 = jnp.sum(w1 * m2w1, axis=0, keepdims=True) + m2b1
    mask1 = jax.nn.sigmoid(s1)
    h1 = mask1 * full1[64:96]
    x2 = jnp.maximum((adots(h1) + full1[96:128]) * mask1, 0.0)

    y = jnp.maximum(_dg(sT[0:_H, 224:256], x2, 1, 0) + p1b, 0.0)
    dT = jnp.concatenate([d_ref[g] for g in range(G)], axis=1)
    y = y * dT
    out = _dg(sT[0:_H, 256:288], y, 1, 0) + p2b
    for g in range(G):
        o_ref[g] = out[:, g * n:(g + 1) * n]


def kernel(x, adj, slab, drop_scale):
    b, n, ci = x.shape
    xt = x.transpose(0, 2, 1)
    dt = drop_scale.transpose(0, 2, 1)
    st = slab.T
    G = next(g for g in (16, 8, 4, 2, 1) if b % g == 0)
    out = pl.pallas_call(
        _smg_body,
        out_shape=jax.ShapeDtypeStruct((b, _H, n), jnp.float32),
        grid=(b // G,),
        in_specs=[
            pl.BlockSpec((G, ci, n), lambda i: (i, 0, 0)),
            pl.BlockSpec((G, n, n), lambda i: (i, 0, 0)),
            pl.BlockSpec((G, _H, n), lambda i: (i, 0, 0)),
            pl.BlockSpec(st.shape, lambda i: (0, 0)),
        ],
        out_specs=pl.BlockSpec((G, _H, n), lambda i: (i, 0, 0)),
        compiler_params=pltpu.CompilerParams(
            dimension_semantics=("parallel",)),
    )(xt, adj, dt, st)
    return out.transpose(0, 2, 1)


# speedup vs baseline: 1.0525x; 1.0525x over previous
import jax
import jax.numpy as jnp
from jax.experimental import pallas as pl
from jax.experimental.pallas import tpu as pltpu

_H = 32


def _dg(a, b, ca, cb):
    return jax.lax.dot_general(
        a, b, (((ca,), (cb,)), ((), ())), preferred_element_type=jnp.float32)


def _smg_body(x_ref, a_ref, d_ref, s_ref, o_ref):
    G = x_ref.shape[0]
    n = x_ref.shape[2]

    sT = s_ref
    bcol = sT[:, 288:368]
    l12b0 = bcol[0:64, 0:1]
    m1b0 = bcol[0:32, 8:9]
    m2w0 = bcol[0:32, 16:17]
    m2b0 = bcol[0:1, 24:25]
    l12b1 = bcol[0:64, 32:33]
    m1b1 = bcol[0:32, 40:41]
    m2w1 = bcol[0:32, 48:49]
    m2b1 = bcol[0:1, 56:57]
    p1b = bcol[0:32, 64:65]
    p2b = bcol[0:32, 72:73]

    AT16 = [jnp.swapaxes(a_ref[g][...], 0, 1).astype(jnp.bfloat16)
            for g in range(G)]

    def adots(lhs):
        lhs = lhs.astype(jnp.bfloat16)
        return jnp.concatenate(
            [_dg(lhs[:, g * n:(g + 1) * n], AT16[g], 1, 0) for g in range(G)],
            axis=1)

    Wp0 = jnp.concatenate([sT[:, 0:16], sT[:, 80:96]], axis=0)
    full0 = jnp.concatenate(
        [_dg(Wp0, x_ref[g], 1, 0) for g in range(G)], axis=1)
    hl0 = full0[0:64] + l12b0
    agg0 = adots(hl0[0:32])
    cat0 = jnp.maximum(jnp.concatenate([agg0, hl0[32:64]], axis=0), 0.0)
    w0 = jnp.maximum(_dg(sT[0:_H, 16:80], cat0, 1, 0) + m1b0, 0.0)
    s0 = jnp.sum(w0 * m2w0, axis=0, keepdims=True) + m2b0
    mask0 = jax.nn.sigmoid(s0)
    h0 = mask0 * full0[64:96]
    x1 = jnp.maximum((adots(h0) + full0[96:128]) * mask0, 0.0)

    Wp1 = jnp.concatenate([sT[:, 96:128], sT[:, 192:224]], axis=0)
    full1 = _dg(Wp1, x1, 1, 0)
    hl1 = full1[0:64] * mask0 + l12b1
    agg1 = adots(hl1[0:32])
    cat1 = jnp.maximum(jnp.concatenate([agg1, hl1[32:64]], axis=0), 0.0)
    w1 = jnp.maximum(_dg(sT[0:_H, 128:192], cat1, 1, 0) + m1b1, 0.0)
    s1 = jnp.sum(w1 * m2w1, axis=0, keepdims=True) + m2b1
    mask1 = jax.nn.sigmoid(s1)
    h1 = mask1 * full1[64:96]
    x2 = jnp.maximum((adots(h1) + full1[96:128]) * mask1, 0.0)

    y = jnp.maximum(_dg(sT[0:_H, 224:256], x2, 1, 0) + p1b, 0.0)
    dT = jnp.concatenate([d_ref[g] for g in range(G)], axis=1)
    y = y * dT
    out = _dg(sT[0:_H, 256:288], y, 1, 0) + p2b
    for g in range(G):
        o_ref[g] = out[:, g * n:(g + 1) * n]


def kernel(x, adj, slab, drop_scale):
    b, n, ci = x.shape
    xt = x.transpose(0, 2, 1)
    dt = drop_scale.transpose(0, 2, 1)
    st = slab.T
    G = next(g for g in (32, 16, 8, 4, 2, 1) if b % g == 0)
    out = pl.pallas_call(
        _smg_body,
        out_shape=jax.ShapeDtypeStruct((b, _H, n), jnp.float32),
        grid=(b // G,),
        in_specs=[
            pl.BlockSpec((G, ci, n), lambda i: (i, 0, 0)),
            pl.BlockSpec((G, n, n), lambda i: (i, 0, 0)),
            pl.BlockSpec((G, _H, n), lambda i: (i, 0, 0)),
            pl.BlockSpec(st.shape, lambda i: (0, 0)),
        ],
        out_specs=pl.BlockSpec((G, _H, n), lambda i: (i, 0, 0)),
        compiler_params=pltpu.CompilerParams(
            dimension_semantics=("parallel",)),
    )(xt, adj, dt, st)
    return out.transpose(0, 2, 1)
